# Initial kernel scaffold; baseline (speedup 1.0000x reference)
import jax
import jax.numpy as jnp
from jax.experimental import pallas as pl


def kernel(x, gate_w, W1, b1, W2, b2, lora_A, lora_B):
    raise NotImplementedError("write your pallas kernel here")


# speedup vs baseline: 1.3100x; 1.3100x over previous
import functools

import jax
import jax.numpy as jnp
from jax.experimental import pallas as pl
from jax.experimental.pallas import tpu as pltpu

D_MODEL = 768
FFN = 3072
NUM_EXPERTS = 64
LORA_RANK = 16
N_TOKENS = 4096
TB = 512
NT = N_TOKENS // TB


def _moe_body(x_ref, gw_ref, w1_ref, b1_ref, w2_ref, b2_ref, a_ref, bm_ref,
              out_ref):
    e = pl.program_id(1)
    xb = x_ref[...]
    gw = gw_ref[...]
    logits = jax.lax.dot_general(xb, gw, (((1,), (1,)), ((), ())))
    m1 = jnp.max(logits, axis=-1, keepdims=True)
    lane = jax.lax.broadcasted_iota(jnp.int32, logits.shape, 1)
    is_max = logits >= m1
    first = jnp.min(jnp.where(is_max, lane, NUM_EXPERTS), axis=-1,
                    keepdims=True)
    l2 = jnp.max(jnp.where(lane == first, -jnp.inf, logits), axis=-1,
                 keepdims=True)
    d = jnp.exp(l2 - m1)
    c = jnp.where(e == 0, 1.0 / (1.0 + d), d / (1.0 + d))
    my_logit = jnp.where(e == 0, logits[:, 0:1], logits[:, 1:2])
    coef = jnp.where(my_logit >= l2, c, 0.0)

    w1 = w1_ref[0]
    h = jax.lax.dot_general(xb, w1, (((1,), (1,)), ((), ()))) + b1_ref[0]
    h = jax.nn.gelu(h)
    base = jax.lax.dot_general(h, w2_ref[0], (((1,), (1,)), ((), ()))) \
        + b2_ref[0]
    t = jax.lax.dot_general(xb, a_ref[0], (((1,), (1,)), ((), ())))
    lora = jax.lax.dot_general(t, bm_ref[0], (((1,), (1,)), ((), ())))
    y = (base + lora) * coef

    @pl.when(e == 0)
    def _():
        out_ref[...] = y

    @pl.when(e == 1)
    def _():
        out_ref[...] += y


@jax.jit
def kernel(x, gate_w, W1, b1, W2, b2, lora_A, lora_B):
    bs, sl, d = x.shape
    x_flat = x.reshape(-1, d)
    b1r = b1.reshape(2, 1, FFN)
    b2r = b2.reshape(2, 1, D_MODEL)
    out = pl.pallas_call(
        _moe_body,
        grid=(NT, 2),
        in_specs=[
            pl.BlockSpec((TB, D_MODEL), lambda t, e: (t, 0)),
            pl.BlockSpec((NUM_EXPERTS, D_MODEL), lambda t, e: (0, 0)),
            pl.BlockSpec((1, FFN, D_MODEL), lambda t, e: (e, 0, 0)),
            pl.BlockSpec((1, 1, FFN), lambda t, e: (e, 0, 0)),
            pl.BlockSpec((1, D_MODEL, FFN), lambda t, e: (e, 0, 0)),
            pl.BlockSpec((1, 1, D_MODEL), lambda t, e: (e, 0, 0)),
            pl.BlockSpec((1, LORA_RANK, D_MODEL), lambda t, e: (e, 0, 0)),
            pl.BlockSpec((1, D_MODEL, LORA_RANK), lambda t, e: (e, 0, 0)),
        ],
        out_specs=pl.BlockSpec((TB, D_MODEL), lambda t, e: (t, 0)),
        out_shape=jax.ShapeDtypeStruct((N_TOKENS, D_MODEL), jnp.float32),
    )(x_flat, gate_w, W1, b1r, W2, b2r, lora_A, lora_B)
    return out.reshape(bs, sl, d)


# speedup vs baseline: 1.5036x; 1.1477x over previous
import functools

import jax
import jax.numpy as jnp
from jax import lax
from jax.experimental import pallas as pl
from jax.experimental.pallas import tpu as pltpu
from jax.experimental.pallas import tpu_sc as plsc

D_MODEL = 768
FFN = 3072
NUM_EXPERTS = 64
LORA_RANK = 16
NTOK = 4096
CAP = NTOK
GB = 1024
NGB = NTOK // GB
TBC = 512
NCH = CAP // TBC
HALF = NTOK // 2
TRASH = NTOK


HI = 64
LO = CAP // HI


def _route_tc_body(x_ref, gw_ref, idx_ref, cc_ref, cnt_ref,
                   carry_ref, cds_ref, poss_ref):
    t = pl.program_id(0)

    @pl.when(t == 0)
    def _():
        carry_ref[0] = 0
        carry_ref[1] = 0

    @pl.when(t < NGB)
    def _():
        xb = x_ref[...]
        gw = gw_ref[...]
        lg = lax.dot_general(gw, xb, (((1,), (1,)), ((), ())))
        m1 = jnp.max(lg, axis=0, keepdims=True)
        row = lax.broadcasted_iota(jnp.int32, lg.shape, 0)
        first = jnp.min(jnp.where(lg >= m1, row, NUM_EXPERTS), axis=0,
                        keepdims=True)
        l2 = jnp.max(jnp.where(row == first, -jnp.inf, lg), axis=0,
                     keepdims=True)
        d = jnp.exp(l2 - m1)
        c0 = jnp.where(lg[0:1, :] >= l2, 1.0 / (1.0 + d), 0.0)
        c1 = jnp.where(lg[1:2, :] >= l2, d / (1.0 + d), 0.0)
        cd = jnp.concatenate([c0, c1], axis=0)

        mf = jnp.where(cd > 0.0, 1.0, 0.0)
        r_i = lax.broadcasted_iota(jnp.int32, (GB, GB), 0)
        c_i = lax.broadcasted_iota(jnp.int32, (GB, GB), 1)
        tri = jnp.where(r_i <= c_i, 1.0, 0.0)
        cs = lax.dot_general(mf, tri,
                             (((1,), (0,)), ((), ()))).astype(jnp.int32)
        ca0 = carry_ref[0]
        ca1 = carry_ref[1]
        carry = jnp.concatenate(
            [jnp.full((1, GB), ca0, jnp.int32),
             jnp.full((1, GB), ca1, jnp.int32)], axis=0)
        cds_ref[t] = cd
        poss_ref[t] = jnp.clip(cs - 1 + carry, 0, CAP - 1)
        tot0 = jnp.sum(mf[0:1, :]).astype(jnp.int32)
        tot1 = jnp.sum(mf[1:2, :]).astype(jnp.int32)
        carry_ref[0] = ca0 + tot0
        carry_ref[1] = ca1 + tot1

    @pl.when(t == NGB - 1)
    def _():
        cnt_ref[...] = jnp.concatenate(
            [jnp.full((1, 16), carry_ref[0], jnp.int32),
             jnp.full((1, 16), carry_ref[1], jnp.int32)], axis=0)

    @pl.when(t >= NGB)
    def _():
        e = t - NGB
        cd_full = jnp.concatenate([cds_ref[g] for g in range(NGB)],
                                  axis=1)
        pos_full = jnp.concatenate([poss_ref[g] for g in range(NGB)],
                                   axis=1)
        cd_r = jnp.where(e == 0, cd_full[0:1], cd_full[1:2])
        pos_r = jnp.where(e == 0, pos_full[0:1], pos_full[1:2])
        routed = cd_r > 0.0
        hi = pos_r // LO
        lo = pos_r - hi * LO
        h_i = lax.broadcasted_iota(jnp.int32, (HI, NTOK), 0)
        l_i = lax.broadcasted_iota(jnp.int32, (LO, NTOK), 0)
        oh_hi = jnp.where((hi == h_i) & routed, 1.0, 0.0)
        oh_lo = jnp.where(lo == l_i, 1.0, 0.0)
        tok = lax.broadcasted_iota(
            jnp.int32, (1, NTOK), 1).astype(jnp.float32)
        prec = lax.Precision.HIGHEST
        idx_m = lax.dot_general(oh_hi, oh_lo * tok,
                                (((1,), (1,)), ((), ())),
                                precision=prec)
        cc_m = lax.dot_general(oh_hi, oh_lo * cd_r,
                               (((1,), (1,)), ((), ())),
                               precision=prec)
        idx_ref[0] = idx_m.astype(jnp.int32)
        cc_ref[0] = cc_m


def _route_tc_call(x_flat, gate_w):
    return pl.pallas_call(
        _route_tc_body,
        grid=(NGB + 2,),
        in_specs=[
            pl.BlockSpec((GB, D_MODEL),
                         lambda t: (jnp.minimum(t, NGB - 1), 0)),
            pl.BlockSpec((NUM_EXPERTS, D_MODEL), lambda t: (0, 0)),
        ],
        out_specs=[
            pl.BlockSpec((1, HI, LO),
                         lambda t: (jnp.maximum(t - NGB, 0), 0, 0)),
            pl.BlockSpec((1, HI, LO),
                         lambda t: (jnp.maximum(t - NGB, 0), 0, 0)),
            pl.BlockSpec((2, 16), lambda t: (0, 0)),
        ],
        out_shape=[
            jax.ShapeDtypeStruct((2, HI, LO), jnp.int32),
            jax.ShapeDtypeStruct((2, HI, LO), jnp.float32),
            jax.ShapeDtypeStruct((2, 16), jnp.int32),
        ],
        scratch_shapes=[
            pltpu.SMEM((2,), jnp.int32),
            pltpu.VMEM((NGB, 2, GB), jnp.float32),
            pltpu.VMEM((NGB, 2, GB), jnp.int32),
        ],
    )(x_flat, gate_w)


def _gather_one(idx_hbm, cnt_hbm, x_hbm, xc_dst, s, cnt_v, idx16_v, rows_v,
                sem):
    pltpu.sync_copy(cnt_hbm, cnt_v)
    cnt = cnt_v[...][0]
    nch = (cnt + 255) // 256

    def chunk(k, carry):
        base = k * 256 + s * 16

        @pl.when(base < cnt)
        def _():
            pltpu.sync_copy(idx_hbm.at[pl.ds(base, 16)], idx16_v)
            tok = jnp.minimum(idx16_v[...], NTOK - 1)
            pltpu.async_copy(x_hbm.at[tok], rows_v, sem).wait()
            pltpu.sync_copy(rows_v, xc_dst.at[pl.ds(base, 16)])

        return carry

    lax.fori_loop(0, nch, chunk, 0)


def _gather_body(idx0_hbm, idx1_hbm, cnt_hbm, x_hbm, xc_hbm,
                 cnt_v, idx16_v, rows_v, sem):
    c = lax.axis_index("c")
    s = lax.axis_index("s")

    @pl.when(c == 0)
    def _():
        _gather_one(idx0_hbm, cnt_hbm.at[0], x_hbm, xc_hbm.at[0], s,
                    cnt_v, idx16_v, rows_v, sem)

    @pl.when(c == 1)
    def _():
        _gather_one(idx1_hbm, cnt_hbm.at[1], x_hbm, xc_hbm.at[1], s,
                    cnt_v, idx16_v, rows_v, sem)


def _gather_call(idx0, idx1, cnt, x_flat):
    mesh = plsc.VectorSubcoreMesh(core_axis_name="c", subcore_axis_name="s")
    return pl.kernel(
        _gather_body,
        out_type=jax.ShapeDtypeStruct((2, CAP, D_MODEL), jnp.float32),
        mesh=mesh,
        scratch_types=[
            pltpu.VMEM((16,), jnp.int32),
            pltpu.VMEM((16,), jnp.int32),
            pltpu.VMEM((16, D_MODEL), jnp.float32),
            pltpu.SemaphoreType.DMA,
        ],
    )(idx0, idx1, cnt, x_flat)


KCH = 2
FK = FFN // KCH


def _ffn_body(cnt_ref, xc_ref, cc_ref, w1_ref, b1_ref, w2_ref, b2_ref,
              a_ref, bm_ref, y_ref, acc_ref):
    e = pl.program_id(0)
    j = pl.program_id(1)
    k = pl.program_id(2)

    @pl.when(j * TBC < cnt_ref[e])
    def _():
        bf = jnp.bfloat16
        f32 = jnp.float32
        xb = xc_ref[0].astype(bf)
        h = lax.dot_general(xb, w1_ref[0].astype(bf),
                            (((1,), (1,)), ((), ())),
                            preferred_element_type=f32) + b1_ref[0]
        h = jax.nn.gelu(h).astype(bf)
        part = lax.dot_general(h, w2_ref[0].astype(bf),
                               (((1,), (1,)), ((), ())),
                               preferred_element_type=f32)

        @pl.when(k == 0)
        def _():
            t = lax.dot_general(xb, a_ref[0].astype(bf),
                                (((1,), (1,)), ((), ())),
                                preferred_element_type=f32).astype(bf)
            lora = lax.dot_general(t, bm_ref[0].astype(bf),
                                   (((1,), (1,)), ((), ())),
                                   preferred_element_type=f32)
            acc_ref[...] = part + b2_ref[0] + lora

        @pl.when(k == KCH - 1)
        def _():
            y_ref[0] = (acc_ref[...] + part) * cc_ref[0]


def _jclamp(e, j, cnt):
    nb = jnp.maximum((cnt[e] + TBC - 1) // TBC, 1)
    return jnp.minimum(j, nb - 1)


def _kclamp(e, j, k, cnt):
    return jnp.where(j * TBC < cnt[e], k, KCH - 1)


def _ffn_call(cnt2, xc, cc3, W1, b1r, W2, b2r, lora_A, lora_B):
    grid_spec = pltpu.PrefetchScalarGridSpec(
        num_scalar_prefetch=1,
        grid=(2, NCH, KCH),
        in_specs=[
            pl.BlockSpec((1, TBC, D_MODEL),
                         lambda e, j, k, cnt: (e, _jclamp(e, j, cnt), 0)),
            pl.BlockSpec((1, TBC, 1),
                         lambda e, j, k, cnt: (e, _jclamp(e, j, cnt), 0)),
            pl.BlockSpec((1, FK, D_MODEL),
                         lambda e, j, k, cnt: (e, _kclamp(e, j, k, cnt), 0)),
            pl.BlockSpec((1, 1, FK),
                         lambda e, j, k, cnt: (e, 0, _kclamp(e, j, k, cnt))),
            pl.BlockSpec((1, D_MODEL, FK),
                         lambda e, j, k, cnt: (e, 0, _kclamp(e, j, k, cnt))),
            pl.BlockSpec((1, 1, D_MODEL), lambda e, j, k, cnt: (e, 0, 0)),
            pl.BlockSpec((1, LORA_RANK, D_MODEL),
                         lambda e, j, k, cnt: (e, 0, 0)),
            pl.BlockSpec((1, D_MODEL, LORA_RANK),
                         lambda e, j, k, cnt: (e, 0, 0)),
        ],
        out_specs=pl.BlockSpec((1, TBC, D_MODEL),
                               lambda e, j, k, cnt: (e, _jclamp(e, j, cnt),
                                                     0)),
        scratch_shapes=[pltpu.VMEM((TBC, D_MODEL), jnp.float32)],
    )
    return pl.pallas_call(
        _ffn_body,
        grid_spec=grid_spec,
        out_shape=jax.ShapeDtypeStruct((2, CAP, D_MODEL), jnp.float32),
    )(cnt2, xc, cc3, W1, b1r, W2, b2r, lora_A, lora_B)


def _combine_body(y_hbm, idx_hbm, cnt_hbm, zr_hbm, out_hbm,
                  zero_v, il0_v, il1_v, cnt0_v, cnt1_v, y0r_v, y1r_v,
                  crows_v, sem):
    c = lax.axis_index("c")
    s = lax.axis_index("s")
    tbase = c * HALF + s * 128
    lo = c * HALF
    cp0 = pltpu.async_copy(zr_hbm, zero_v, sem)
    cp1 = pltpu.async_copy(idx_hbm.at[0], il0_v, sem)
    cp2 = pltpu.async_copy(idx_hbm.at[1], il1_v, sem)
    cp3 = pltpu.async_copy(cnt_hbm.at[0], cnt0_v, sem)
    cp4 = pltpu.async_copy(cnt_hbm.at[1], cnt1_v, sem)
    cp0.wait()
    zs = [pltpu.async_copy(zero_v, out_hbm.at[pl.ds(tbase + 16 * r, 16)],
                           sem) for r in range(8)]
    jc0 = jnp.minimum(lax.iota(jnp.int32, 16) + jnp.full((16,), s * 16,
                                                         jnp.int32),
                      CAP - 1)
    g0 = pltpu.async_copy(y_hbm.at[0].at[jc0], y0r_v, sem)
    g1 = pltpu.async_copy(y_hbm.at[1].at[jc0], y1r_v, sem)
    cp1.wait()
    cp2.wait()
    cp3.wait()
    cp4.wait()
    for z in zs:
        z.wait()
    g0.wait()
    g1.wait()
    plsc.subcore_barrier()

    def handle(e, base, cnt, il_v, rows_v):
        j16 = lax.iota(jnp.int32, 16) + jnp.full((16,), base, jnp.int32)
        valid = j16 < jnp.full((16,), cnt, jnp.int32)
        tok = il_v[pl.ds(base, 16)]
        inh = valid & (tok >= lo) & (tok < lo + HALF)
        sidx = jnp.where(inh, tok, jnp.full((16,), TRASH, jnp.int32) + s)
        if e == 0:
            pltpu.async_copy(rows_v, out_hbm.at[sidx], sem).wait()
        else:
            pltpu.async_copy(out_hbm.at[sidx], crows_v, sem).wait()
            for r in range(16):

                def addrow(v, carry2, r=r):
                    crows_v[r, pl.ds(v * 16, 16)] = (
                        crows_v[r, pl.ds(v * 16, 16)]
                        + rows_v[r, pl.ds(v * 16, 16)])
                    return carry2

                lax.fori_loop(0, D_MODEL // 16, addrow, 0, unroll=8)
            pltpu.async_copy(crows_v, out_hbm.at[sidx], sem).wait()

    for e, il_v, cnt_v, rows_v in ((0, il0_v, cnt0_v, y0r_v),
                                   (1, il1_v, cnt1_v, y1r_v)):
        cnt = cnt_v[...][0]

        @pl.when(s * 16 < cnt)
        def _(e=e, il_v=il_v, cnt=cnt, rows_v=rows_v):
            handle(e, s * 16, cnt, il_v, rows_v)

        ngrp = (cnt + 255) // 256

        def grp(n, carry, e=e, il_v=il_v, cnt=cnt, rows_v=rows_v):
            base = n * 256 + s * 16

            @pl.when(base < cnt)
            def _():
                j16 = lax.iota(jnp.int32, 16) + jnp.full((16,), base,
                                                         jnp.int32)
                jc = jnp.minimum(j16, CAP - 1)
                pltpu.async_copy(y_hbm.at[e].at[jc], rows_v, sem).wait()
                handle(e, base, cnt, il_v, rows_v)

            return carry

        lax.fori_loop(1, ngrp, grp, 0)
        if e == 0:
            plsc.subcore_barrier()


def _combine_call(y, idx, cnt, zr):
    mesh = plsc.VectorSubcoreMesh(core_axis_name="c", subcore_axis_name="s")
    return pl.kernel(
        _combine_body,
        out_type=jax.ShapeDtypeStruct((NTOK + 16, D_MODEL), jnp.float32),
        mesh=mesh,
        scratch_types=[
            pltpu.VMEM((16, D_MODEL), jnp.float32),
            pltpu.VMEM((CAP,), jnp.int32),
            pltpu.VMEM((CAP,), jnp.int32),
            pltpu.VMEM((16,), jnp.int32),
            pltpu.VMEM((16,), jnp.int32),
            pltpu.VMEM((16, D_MODEL), jnp.float32),
            pltpu.VMEM((16, D_MODEL), jnp.float32),
            pltpu.VMEM((16, D_MODEL), jnp.float32),
            pltpu.SemaphoreType.DMA,
        ],
    )(y, idx, cnt, zr)


@jax.jit
def kernel(x, gate_w, W1, b1, W2, b2, lora_A, lora_B):
    bs, sl, d = x.shape
    x_flat = x.reshape(-1, d)
    b1r = b1.reshape(2, 1, FFN)
    b2r = b2.reshape(2, 1, D_MODEL)
    idx_m, cc_m, cnt = _route_tc_call(x_flat, gate_w)
    idx = idx_m.reshape(2, CAP)
    xc = _gather_call(idx[0], idx[1], cnt, x_flat)
    cc3 = cc_m.reshape(2, CAP, 1)
    y = _ffn_call(cnt[:, 0], xc, cc3, W1, b1r, W2, b2r, lora_A, lora_B)
    zr = jnp.zeros((16, D_MODEL), jnp.float32)
    out = _combine_call(y, idx, cnt, zr)
    return out[:NTOK].reshape(bs, sl, d)


# speedup vs baseline: 1.5468x; 1.0288x over previous
import functools

import jax
import jax.numpy as jnp
from jax import lax
from jax.experimental import pallas as pl
from jax.experimental.pallas import tpu as pltpu
from jax.experimental.pallas import tpu_sc as plsc

D_MODEL = 768
FFN = 3072
NUM_EXPERTS = 64
LORA_RANK = 16
NTOK = 4096
CAP = NTOK
GB = 1024
NGB = NTOK // GB
TBC = 512
NCH = CAP // TBC
HALF = NTOK // 2
TRASH = NTOK


HI = 64
LO = CAP // HI


def _route_tc_body(x_ref, gw_ref, idx_ref, cc_ref, cnt_ref,
                   carry_ref, cds_ref, poss_ref):
    t = pl.program_id(0)

    @pl.when(t == 0)
    def _():
        carry_ref[0] = 0
        carry_ref[1] = 0

    @pl.when(t < NGB)
    def _():
        xb = x_ref[...]
        gw = gw_ref[...]
        lg = lax.dot_general(gw, xb, (((1,), (1,)), ((), ())))
        m1 = jnp.max(lg, axis=0, keepdims=True)
        row = lax.broadcasted_iota(jnp.int32, lg.shape, 0)
        first = jnp.min(jnp.where(lg >= m1, row, NUM_EXPERTS), axis=0,
                        keepdims=True)
        l2 = jnp.max(jnp.where(row == first, -jnp.inf, lg), axis=0,
                     keepdims=True)
        d = jnp.exp(l2 - m1)
        c0 = jnp.where(lg[0:1, :] >= l2, 1.0 / (1.0 + d), 0.0)
        c1 = jnp.where(lg[1:2, :] >= l2, d / (1.0 + d), 0.0)
        cd = jnp.concatenate([c0, c1], axis=0)

        mf = jnp.where(cd > 0.0, 1.0, 0.0)
        r_i = lax.broadcasted_iota(jnp.int32, (GB, GB), 0)
        c_i = lax.broadcasted_iota(jnp.int32, (GB, GB), 1)
        tri = jnp.where(r_i <= c_i, 1.0, 0.0)
        cs = lax.dot_general(mf, tri,
                             (((1,), (0,)), ((), ()))).astype(jnp.int32)
        ca0 = carry_ref[0]
        ca1 = carry_ref[1]
        carry = jnp.concatenate(
            [jnp.full((1, GB), ca0, jnp.int32),
             jnp.full((1, GB), ca1, jnp.int32)], axis=0)
        cds_ref[t] = cd
        poss_ref[t] = jnp.clip(cs - 1 + carry, 0, CAP - 1)
        tot0 = jnp.sum(mf[0:1, :]).astype(jnp.int32)
        tot1 = jnp.sum(mf[1:2, :]).astype(jnp.int32)
        carry_ref[0] = ca0 + tot0
        carry_ref[1] = ca1 + tot1

    @pl.when(t == NGB - 1)
    def _():
        cnt_ref[...] = jnp.concatenate(
            [jnp.full((1, 16), carry_ref[0], jnp.int32),
             jnp.full((1, 16), carry_ref[1], jnp.int32)], axis=0)

    @pl.when(t >= NGB)
    def _():
        e = t - NGB
        cd_full = jnp.concatenate([cds_ref[g] for g in range(NGB)],
                                  axis=1)
        pos_full = jnp.concatenate([poss_ref[g] for g in range(NGB)],
                                   axis=1)
        cd_r = jnp.where(e == 0, cd_full[0:1], cd_full[1:2])
        pos_r = jnp.where(e == 0, pos_full[0:1], pos_full[1:2])
        routed = cd_r > 0.0
        hi = pos_r // LO
        lo = pos_r - hi * LO
        h_i = lax.broadcasted_iota(jnp.int32, (HI, NTOK), 0)
        l_i = lax.broadcasted_iota(jnp.int32, (LO, NTOK), 0)
        oh_hi = jnp.where((hi == h_i) & routed, 1.0, 0.0)
        oh_lo = jnp.where(lo == l_i, 1.0, 0.0)
        tok = lax.broadcasted_iota(
            jnp.int32, (1, NTOK), 1).astype(jnp.float32)
        prec = lax.Precision.HIGHEST
        idx_m = lax.dot_general(oh_hi, oh_lo * tok,
                                (((1,), (1,)), ((), ())),
                                precision=prec)
        cc_m = lax.dot_general(oh_hi, oh_lo * cd_r,
                               (((1,), (1,)), ((), ())),
                               precision=prec)
        idx_ref[0] = idx_m.astype(jnp.int32)
        cc_ref[0] = cc_m


def _route_tc_call(x_flat, gate_w):
    return pl.pallas_call(
        _route_tc_body,
        grid=(NGB + 2,),
        in_specs=[
            pl.BlockSpec((GB, D_MODEL),
                         lambda t: (jnp.minimum(t, NGB - 1), 0)),
            pl.BlockSpec((NUM_EXPERTS, D_MODEL), lambda t: (0, 0)),
        ],
        out_specs=[
            pl.BlockSpec((1, HI, LO),
                         lambda t: (jnp.maximum(t - NGB, 0), 0, 0)),
            pl.BlockSpec((1, HI, LO),
                         lambda t: (jnp.maximum(t - NGB, 0), 0, 0)),
            pl.BlockSpec((2, 16), lambda t: (0, 0)),
        ],
        out_shape=[
            jax.ShapeDtypeStruct((2, HI, LO), jnp.int32),
            jax.ShapeDtypeStruct((2, HI, LO), jnp.float32),
            jax.ShapeDtypeStruct((2, 16), jnp.int32),
        ],
        scratch_shapes=[
            pltpu.SMEM((2,), jnp.int32),
            pltpu.VMEM((NGB, 2, GB), jnp.float32),
            pltpu.VMEM((NGB, 2, GB), jnp.int32),
        ],
    )(x_flat, gate_w)


def _gather_one(idx_hbm, cnt_hbm, x_hbm, xc_dst, s, cnt_v, idx16_v, rows_v,
                sem):
    pltpu.sync_copy(cnt_hbm, cnt_v)
    cnt = cnt_v[...][0]
    nch = (cnt + 255) // 256

    def chunk(k, carry):
        base = k * 256 + s * 16

        @pl.when(base < cnt)
        def _():
            pltpu.sync_copy(idx_hbm.at[pl.ds(base, 16)], idx16_v)
            tok = jnp.minimum(idx16_v[...], NTOK - 1)
            pltpu.async_copy(x_hbm.at[tok], rows_v, sem).wait()
            pltpu.sync_copy(rows_v, xc_dst.at[pl.ds(base, 16)])

        return carry

    lax.fori_loop(0, nch, chunk, 0)


def _gather_body(idx0_hbm, idx1_hbm, cnt_hbm, x_hbm, xc_hbm,
                 cnt_v, idx16_v, rows_v, sem):
    c = lax.axis_index("c")
    s = lax.axis_index("s")

    @pl.when(c == 0)
    def _():
        _gather_one(idx0_hbm, cnt_hbm.at[0], x_hbm, xc_hbm.at[0], s,
                    cnt_v, idx16_v, rows_v, sem)

    @pl.when(c == 1)
    def _():
        _gather_one(idx1_hbm, cnt_hbm.at[1], x_hbm, xc_hbm.at[1], s,
                    cnt_v, idx16_v, rows_v, sem)


def _gather_call(idx0, idx1, cnt, x_flat):
    mesh = plsc.VectorSubcoreMesh(core_axis_name="c", subcore_axis_name="s")
    return pl.kernel(
        _gather_body,
        out_type=jax.ShapeDtypeStruct((2, CAP, D_MODEL), jnp.float32),
        mesh=mesh,
        scratch_types=[
            pltpu.VMEM((16,), jnp.int32),
            pltpu.VMEM((16,), jnp.int32),
            pltpu.VMEM((16, D_MODEL), jnp.float32),
            pltpu.SemaphoreType.DMA,
        ],
    )(idx0, idx1, cnt, x_flat)


KCH = 2
FK = FFN // KCH


def _ffn_body(cnt_ref, xc_ref, cc_ref, w1_ref, b1_ref, w2_ref, b2_ref,
              a_ref, bm_ref, y_ref, acc_ref):
    e = pl.program_id(0)
    j = pl.program_id(1)
    k = pl.program_id(2)

    @pl.when(j * TBC < cnt_ref[e])
    def _():
        bf = jnp.bfloat16
        f32 = jnp.float32
        xb = xc_ref[0].astype(bf)
        h = lax.dot_general(xb, w1_ref[0].astype(bf),
                            (((1,), (1,)), ((), ())),
                            preferred_element_type=f32) + b1_ref[0]
        h = jax.nn.gelu(h).astype(bf)
        part = lax.dot_general(h, w2_ref[0].astype(bf),
                               (((1,), (1,)), ((), ())),
                               preferred_element_type=f32)

        @pl.when(k == 0)
        def _():
            t = lax.dot_general(xb, a_ref[0].astype(bf),
                                (((1,), (1,)), ((), ())),
                                preferred_element_type=f32).astype(bf)
            lora = lax.dot_general(t, bm_ref[0].astype(bf),
                                   (((1,), (1,)), ((), ())),
                                   preferred_element_type=f32)
            acc_ref[...] = part + b2_ref[0] + lora

        @pl.when(k == KCH - 1)
        def _():
            y_ref[0] = (acc_ref[...] + part) * cc_ref[0]


def _jclamp(e, j, cnt):
    nb = jnp.maximum((cnt[e] + TBC - 1) // TBC, 1)
    return jnp.minimum(j, nb - 1)


def _kclamp(e, j, k, cnt):
    return jnp.where(j * TBC < cnt[e], k, KCH - 1)


def _ffn_call(cnt2, xc, cc3, W1, b1r, W2, b2r, lora_A, lora_B):
    grid_spec = pltpu.PrefetchScalarGridSpec(
        num_scalar_prefetch=1,
        grid=(2, NCH, KCH),
        in_specs=[
            pl.BlockSpec((1, TBC, D_MODEL),
                         lambda e, j, k, cnt: (e, _jclamp(e, j, cnt), 0)),
            pl.BlockSpec((1, TBC, 1),
                         lambda e, j, k, cnt: (e, _jclamp(e, j, cnt), 0)),
            pl.BlockSpec((1, FK, D_MODEL),
                         lambda e, j, k, cnt: (e, _kclamp(e, j, k, cnt), 0)),
            pl.BlockSpec((1, 1, FK),
                         lambda e, j, k, cnt: (e, 0, _kclamp(e, j, k, cnt))),
            pl.BlockSpec((1, D_MODEL, FK),
                         lambda e, j, k, cnt: (e, 0, _kclamp(e, j, k, cnt))),
            pl.BlockSpec((1, 1, D_MODEL), lambda e, j, k, cnt: (e, 0, 0)),
            pl.BlockSpec((1, LORA_RANK, D_MODEL),
                         lambda e, j, k, cnt: (e, 0, 0)),
            pl.BlockSpec((1, D_MODEL, LORA_RANK),
                         lambda e, j, k, cnt: (e, 0, 0)),
        ],
        out_specs=pl.BlockSpec((1, TBC, D_MODEL),
                               lambda e, j, k, cnt: (e, _jclamp(e, j, cnt),
                                                     0)),
        scratch_shapes=[pltpu.VMEM((TBC, D_MODEL), jnp.float32)],
    )
    return pl.pallas_call(
        _ffn_body,
        grid_spec=grid_spec,
        out_shape=jax.ShapeDtypeStruct((2, CAP, D_MODEL), jnp.float32),
    )(cnt2, xc, cc3, W1, b1r, W2, b2r, lora_A, lora_B)


def _combine_body(y_hbm, idx_hbm, cnt_hbm, zr_hbm, out_hbm,
                  zero_v, il0_v, il1_v, cnt0_v, cnt1_v, yrows_v, crows_v,
                  sem):
    c = lax.axis_index("c")
    s = lax.axis_index("s")
    tbase = c * HALF + s * 128
    cp0 = pltpu.async_copy(zr_hbm, zero_v, sem)
    cp1 = pltpu.async_copy(idx_hbm.at[0], il0_v, sem)
    cp2 = pltpu.async_copy(idx_hbm.at[1], il1_v, sem)
    cp3 = pltpu.async_copy(cnt_hbm.at[0], cnt0_v, sem)
    cp4 = pltpu.async_copy(cnt_hbm.at[1], cnt1_v, sem)
    cp0.wait()
    zs = [pltpu.async_copy(zero_v, out_hbm.at[pl.ds(tbase + 16 * r, 16)],
                           sem) for r in range(8)]
    cp1.wait()
    cp2.wait()
    cp3.wait()
    cp4.wait()
    for z in zs:
        z.wait()
    plsc.subcore_barrier()

    lo = c * HALF
    for e, il_v, cnt_v in ((0, il0_v, cnt0_v), (1, il1_v, cnt1_v)):
        cnt = cnt_v[...][0]
        ngrp = (cnt + 255) // 256

        def grp(n, carry, e=e, il_v=il_v, cnt=cnt):
            base = n * 256 + s * 16

            @pl.when(base < cnt)
            def _():
                j16 = lax.iota(jnp.int32, 16) + jnp.full((16,), base,
                                                         jnp.int32)
                valid = j16 < jnp.full((16,), cnt, jnp.int32)
                jc = jnp.minimum(j16, CAP - 1)
                tok = il_v[pl.ds(base, 16)]
                inh = valid & (tok >= lo) & (tok < lo + HALF)
                sidx = jnp.where(inh, tok,
                                 jnp.full((16,), TRASH, jnp.int32) + s)
                pltpu.async_copy(y_hbm.at[e].at[jc], yrows_v, sem).wait()
                if e == 0:
                    pltpu.async_copy(yrows_v, out_hbm.at[sidx], sem).wait()
                else:
                    pltpu.async_copy(out_hbm.at[sidx], crows_v, sem).wait()
                    for r in range(16):

                        def addrow(v, carry2, r=r):
                            crows_v[r, pl.ds(v * 16, 16)] = (
                                crows_v[r, pl.ds(v * 16, 16)]
                                + yrows_v[r, pl.ds(v * 16, 16)])
                            return carry2

                        lax.fori_loop(0, D_MODEL // 16, addrow, 0,
                                      unroll=8)
                    pltpu.async_copy(crows_v, out_hbm.at[sidx],
                                     sem).wait()

            return carry

        lax.fori_loop(0, ngrp, grp, 0)
        if e == 0:
            plsc.subcore_barrier()


def _combine_call(y, idx, cnt, zr):
    mesh = plsc.VectorSubcoreMesh(core_axis_name="c", subcore_axis_name="s")
    return pl.kernel(
        _combine_body,
        out_type=jax.ShapeDtypeStruct((NTOK + 16, D_MODEL), jnp.float32),
        mesh=mesh,
        scratch_types=[
            pltpu.VMEM((16, D_MODEL), jnp.float32),
            pltpu.VMEM((CAP,), jnp.int32),
            pltpu.VMEM((CAP,), jnp.int32),
            pltpu.VMEM((16,), jnp.int32),
            pltpu.VMEM((16,), jnp.int32),
            pltpu.VMEM((16, D_MODEL), jnp.float32),
            pltpu.VMEM((16, D_MODEL), jnp.float32),
            pltpu.SemaphoreType.DMA,
        ],
    )(y, idx, cnt, zr)


@jax.jit
def kernel(x, gate_w, W1, b1, W2, b2, lora_A, lora_B):
    bs, sl, d = x.shape
    x_flat = x.reshape(-1, d)
    b1r = b1.reshape(2, 1, FFN)
    b2r = b2.reshape(2, 1, D_MODEL)
    idx_m, cc_m, cnt = _route_tc_call(x_flat, gate_w)
    idx = idx_m.reshape(2, CAP)
    xc = _gather_call(idx[0], idx[1], cnt, x_flat)
    cc3 = cc_m.reshape(2, CAP, 1)
    y = _ffn_call(cnt[:, 0], xc, cc3, W1, b1r, W2, b2r, lora_A, lora_B)
    zr = jnp.zeros((16, D_MODEL), jnp.float32)
    out = _combine_call(y, idx, cnt, zr)
    return out[:NTOK].reshape(bs, sl, d)
